# baseline (speedup 1.0000x reference)
import functools

import jax
import jax.numpy as jnp
from jax import lax
from jax.experimental import pallas as pl
from jax.experimental.pallas import tpu as pltpu
from jax.experimental.pallas import tpu_sc as plsc

NC, NS, L = 2, 16, 16
NW = NC * NS
B = 16384
D = 128
PW = B // NW
CH = 128
NCH = PW // CH

_BITREV4 = [int(f"{i:04b}"[::-1], 2) for i in range(16)]


def _neg_sqrt(x):
    xs = jnp.maximum(x, jnp.float32(1e-30))
    i = lax.bitcast_convert_type(xs, jnp.int32)
    i = jnp.int32(0x5F3759DF) - (i >> 1)
    y = lax.bitcast_convert_type(i, jnp.float32)
    for _ in range(3):
        y = y * (jnp.float32(1.5) - jnp.float32(0.5) * xs * y * y)
    return -(xs * y)


_GATHER_DNUMS = lax.GatherDimensionNumbers(
    offset_dims=(), collapsed_slice_dims=(0,), start_index_map=(0,))


def _lane_take(v, idx):
    return lax.gather(v, idx[:, None], _GATHER_DNUMS, (1,),
                      mode=lax.GatherScatterMode.PROMISE_IN_BOUNDS)


def _butterfly_sum(vs, lanes):
    h = L // 2
    while len(vs) > 1:
        perm = lanes ^ h
        keep = (lanes & h) == 0
        nxt = []
        for i in range(0, len(vs), 2):
            t1 = vs[i] + _lane_take(vs[i], perm)
            t2 = vs[i + 1] + _lane_take(vs[i + 1], perm)
            nxt.append(jnp.where(keep, t1, t2))
        vs = nxt
        h //= 2
    return vs[0]


def _make_sc_kernel():
    mesh = plsc.VectorSubcoreMesh(core_axis_name="c", subcore_axis_name="s")

    @functools.partial(
        pl.kernel,
        mesh=mesh,
        out_type=jax.ShapeDtypeStruct((B,), jnp.float32),
        scratch_types=[
            pltpu.VMEM((NCH, CH), jnp.int32),
            pltpu.VMEM((NCH, CH), jnp.int32),
            pltpu.VMEM((2, CH, D), jnp.float32),
            pltpu.VMEM((2, CH, D), jnp.float32),
            pltpu.VMEM((PW,), jnp.float32),
            pltpu.SemaphoreType.DMA,
            pltpu.SemaphoreType.DMA,
        ],
    )
    def k(table_hbm, idx1_hbm, idx2_hbm, out_hbm,
          idx1_v, idx2_v, rows1_v, rows2_v, out_v, sem0, sem1):
        wid = lax.axis_index("s") * NC + lax.axis_index("c")
        rbase = wid * NCH
        ic1 = pltpu.async_copy(idx1_hbm.at[pl.ds(rbase, NCH)], idx1_v, sem0)
        ic2 = pltpu.async_copy(idx2_hbm.at[pl.ds(rbase, NCH)], idx2_v, sem0)
        ic1.wait()
        ic2.wait()

        lanes = lax.iota(jnp.int32, L)
        sems = (sem0, sem1)

        def start(c):
            buf, sem = c % 2, sems[c % 2]
            cp1 = pltpu.async_copy(table_hbm.at[idx1_v.at[c]],
                                   rows1_v.at[buf], sem)
            cp2 = pltpu.async_copy(table_hbm.at[idx2_v.at[c]],
                                   rows2_v.at[buf], sem)
            return cp1, cp2

        pend = start(0)
        for c in range(NCH):
            cp1, cp2 = pend
            cp1.wait()
            cp2.wait()
            if c + 1 < NCH:
                pend = start(c + 1)
            buf = c % 2

            @plsc.parallel_loop(0, CH // L, unroll=2)
            def group_body(g, c=c, buf=buf):
                vs = []
                for j in _BITREV4:
                    p = g * L + j
                    acc = [None, None]
                    for s in range(D // L):
                        a = rows1_v[buf, p, pl.ds(s * L, L)]
                        b = rows2_v[buf, p, pl.ds(s * L, L)]
                        df = a - b
                        sq = df * df
                        w = s % 2
                        acc[w] = sq if acc[w] is None else acc[w] + sq
                    vs.append(acc[0] + acc[1])
                ss = _butterfly_sum(vs, lanes)
                out_v[pl.ds(c * CH + g * L, L)] = _neg_sqrt(ss)

        pltpu.sync_copy(out_v, out_hbm.at[pl.ds(wid * PW, PW)])

    return k


_sc_kernel = _make_sc_kernel()


def kernel(ent_emb, rel_emb, sample):
    del rel_emb
    idx1 = sample[:, 0].reshape(B // CH, CH)
    idx2 = sample[:, 1].reshape(B // CH, CH)
    return _sc_kernel(ent_emb, idx1, idx2)


# speedup vs baseline: 1.0224x; 1.0224x over previous
import functools

import jax
import jax.numpy as jnp
from jax import lax
from jax.experimental import pallas as pl
from jax.experimental.pallas import tpu as pltpu
from jax.experimental.pallas import tpu_sc as plsc

NC, NS, L = 2, 16, 16
NW = NC * NS
B = 16384
D = 128
PW = B // NW
CH = 128
NCH = PW // CH

_BITREV4 = [int(f"{i:04b}"[::-1], 2) for i in range(16)]


def _neg_sqrt(x):
    xs = jnp.maximum(x, jnp.float32(1e-30))
    i = lax.bitcast_convert_type(xs, jnp.int32)
    i = jnp.int32(0x5F3759DF) - (i >> 1)
    y = lax.bitcast_convert_type(i, jnp.float32)
    for _ in range(3):
        y = y * (jnp.float32(1.5) - jnp.float32(0.5) * xs * y * y)
    return -(xs * y)


_GATHER_DNUMS = lax.GatherDimensionNumbers(
    offset_dims=(), collapsed_slice_dims=(0,), start_index_map=(0,))


def _lane_take(v, idx):
    return lax.gather(v, idx[:, None], _GATHER_DNUMS, (1,),
                      mode=lax.GatherScatterMode.PROMISE_IN_BOUNDS)


def _butterfly_push(stack, v, lanes):
    lvl = 0
    while stack and stack[-1][0] == lvl:
        _, u = stack.pop()
        h = (L // 2) >> lvl
        perm = lanes ^ h
        keep = (lanes & h) == 0
        v = jnp.where(keep, u + _lane_take(u, perm), v + _lane_take(v, perm))
        lvl += 1
    stack.append((lvl, v))


def _make_sc_kernel():
    mesh = plsc.VectorSubcoreMesh(core_axis_name="c", subcore_axis_name="s")

    @functools.partial(
        pl.kernel,
        mesh=mesh,
        out_type=jax.ShapeDtypeStruct((B,), jnp.float32),
        scratch_types=[
            pltpu.VMEM((NCH, CH), jnp.int32),
            pltpu.VMEM((NCH, CH), jnp.int32),
            pltpu.VMEM((2, CH, D), jnp.float32),
            pltpu.VMEM((2, CH, D), jnp.float32),
            pltpu.VMEM((PW,), jnp.float32),
            pltpu.SemaphoreType.DMA,
            pltpu.SemaphoreType.DMA,
        ],
    )
    def k(table_hbm, idx1_hbm, idx2_hbm, out_hbm,
          idx1_v, idx2_v, rows1_v, rows2_v, out_v, sem0, sem1):
        wid = lax.axis_index("s") * NC + lax.axis_index("c")
        rbase = wid * NCH
        ic1 = pltpu.async_copy(idx1_hbm.at[pl.ds(rbase, NCH)], idx1_v, sem0)
        ic2 = pltpu.async_copy(idx2_hbm.at[pl.ds(rbase, NCH)], idx2_v, sem0)
        ic1.wait()
        ic2.wait()

        lanes = lax.iota(jnp.int32, L)
        sems = (sem0, sem1)

        def start(c):
            buf, sem = c % 2, sems[c % 2]
            cp1 = pltpu.async_copy(table_hbm.at[idx1_v.at[c]],
                                   rows1_v.at[buf], sem)
            cp2 = pltpu.async_copy(table_hbm.at[idx2_v.at[c]],
                                   rows2_v.at[buf], sem)
            return cp1, cp2

        pend = start(0)
        for c in range(NCH):
            cp1, cp2 = pend
            cp1.wait()
            cp2.wait()
            if c + 1 < NCH:
                pend = start(c + 1)
            buf = c % 2

            @plsc.parallel_loop(0, CH // L, unroll=2)
            def group_body(g, c=c, buf=buf):
                stack = []
                for j in _BITREV4:
                    p = g * L + j
                    acc = [None, None]
                    for s in range(D // L):
                        a = rows1_v[buf, p, pl.ds(s * L, L)]
                        b = rows2_v[buf, p, pl.ds(s * L, L)]
                        df = a - b
                        sq = df * df
                        w = s % 2
                        acc[w] = sq if acc[w] is None else acc[w] + sq
                    _butterfly_push(stack, acc[0] + acc[1], lanes)
                ss = stack[0][1]
                out_v[pl.ds(c * CH + g * L, L)] = _neg_sqrt(ss)

        pltpu.sync_copy(out_v, out_hbm.at[pl.ds(wid * PW, PW)])

    return k


_sc_kernel = _make_sc_kernel()


def kernel(ent_emb, rel_emb, sample):
    del rel_emb
    idx1 = sample[:, 0].reshape(B // CH, CH)
    idx2 = sample[:, 1].reshape(B // CH, CH)
    return _sc_kernel(ent_emb, idx1, idx2)
